# baseline (speedup 1.0000x reference)
import numpy as np
import jax
import jax.numpy as jnp
from jax.experimental import pallas as pl
from jax.experimental.pallas import tpu as pltpu

_DIFFUSION_STEPS = 1000
_BETA_START = 0.0001
_BETA_END = 0.02


def _make_tables():
    betas = np.linspace(_BETA_START, _BETA_END, _DIFFUSION_STEPS, dtype=np.float32)
    alphas = (np.float32(1.0) - betas).astype(np.float32)
    alphas_cumprod = np.cumprod(alphas, dtype=np.float32)
    sqrt_acp = np.sqrt(alphas_cumprod).astype(np.float32)
    sqrt_omacp = np.sqrt((np.float32(1.0) - alphas_cumprod)).astype(np.float32)
    return sqrt_acp, sqrt_omacp


_SQRT_ACP, _SQRT_OMACP = _make_tables()

_SAMPLES_PER_BLOCK = 8


def _noise_body(ts_ref, a_tab_ref, b_tab_ref, x_ref, n_ref, n_any_ref,
                o_ref, nout_ref, sem):
    i = pl.program_id(0)
    nsteps = pl.num_programs(0)

    @pl.when(i == 0)
    def _start_passthrough():
        pltpu.make_async_copy(n_any_ref, nout_ref, sem).start()

    for s in range(_SAMPLES_PER_BLOCK):
        t = ts_ref[i * _SAMPLES_PER_BLOCK + s]
        a = a_tab_ref[t]
        b = b_tab_ref[t]
        o_ref[s] = a * x_ref[s] + b * n_ref[s]

    @pl.when(i == nsteps - 1)
    def _finish_passthrough():
        pltpu.make_async_copy(n_any_ref, nout_ref, sem).wait()


def kernel(clean_future, timesteps, noise):
    batch, ch, h, w = clean_future.shape

    spb = _SAMPLES_PER_BLOCK
    block = (spb, ch, h, w)
    grid_spec = pltpu.PrefetchScalarGridSpec(
        num_scalar_prefetch=3,
        grid=(batch // spb,),
        in_specs=[
            pl.BlockSpec(block, lambda i, *_: (i, 0, 0, 0)),
            pl.BlockSpec(block, lambda i, *_: (i, 0, 0, 0)),
            pl.BlockSpec(memory_space=pltpu.MemorySpace.HBM),
        ],
        out_specs=[
            pl.BlockSpec(block, lambda i, *_: (i, 0, 0, 0)),
            pl.BlockSpec(memory_space=pltpu.MemorySpace.HBM),
        ],
        scratch_shapes=[pltpu.SemaphoreType.DMA],
    )

    out, n_out = pl.pallas_call(
        _noise_body,
        grid_spec=grid_spec,
        out_shape=[
            jax.ShapeDtypeStruct(clean_future.shape, jnp.float32),
            jax.ShapeDtypeStruct(clean_future.shape, jnp.float32),
        ],
    )(timesteps, jnp.asarray(_SQRT_ACP), jnp.asarray(_SQRT_OMACP),
      clean_future, noise, noise)

    return out, n_out


# speedup vs baseline: 22.0074x; 22.0074x over previous
import numpy as np
import jax
import jax.numpy as jnp
from jax import lax
from jax.experimental import pallas as pl
from jax.experimental.pallas import tpu as pltpu

_DIFFUSION_STEPS = 1000
_BETA_START = 0.0001
_BETA_END = 0.02


def _make_tables():
    betas = np.linspace(_BETA_START, _BETA_END, _DIFFUSION_STEPS, dtype=np.float32)
    alphas = (np.float32(1.0) - betas).astype(np.float32)
    alphas_cumprod = np.cumprod(alphas, dtype=np.float32)
    sqrt_acp = np.sqrt(alphas_cumprod).astype(np.float32)
    sqrt_omacp = np.sqrt((np.float32(1.0) - alphas_cumprod)).astype(np.float32)
    return sqrt_acp, sqrt_omacp


_SQRT_ACP, _SQRT_OMACP = _make_tables()

_NSLOT = 4


def _body(ts_ref, a_tab_ref, b_tab_ref, x_hbm, n_hbm, o_hbm, no_hbm,
          xb, nb, six, sin_, sox, son):
    i = pl.program_id(0)
    nchunks = x_hbm.shape[0]
    slot = lax.rem(i, _NSLOT)

    @pl.when(i < nchunks)
    def _issue_in():
        @pl.when(i >= _NSLOT)
        def _drain_prev_out():
            pltpu.make_async_copy(xb.at[slot], o_hbm.at[i - _NSLOT], sox.at[slot]).wait()
            pltpu.make_async_copy(nb.at[slot], no_hbm.at[i - _NSLOT], son.at[slot]).wait()

        pltpu.make_async_copy(x_hbm.at[i], xb.at[slot], six.at[slot]).start()
        pltpu.make_async_copy(n_hbm.at[i], nb.at[slot], sin_.at[slot]).start()

    c = i - 2

    @pl.when((c >= 0) & (c < nchunks))
    def _compute():
        cs = lax.rem(c, _NSLOT)
        pltpu.make_async_copy(x_hbm.at[c], xb.at[cs], six.at[cs]).wait()
        pltpu.make_async_copy(n_hbm.at[c], nb.at[cs], sin_.at[cs]).wait()
        t = ts_ref[c]
        a = a_tab_ref[t]
        b = b_tab_ref[t]
        xb[cs] = a * xb[cs] + b * nb[cs]
        pltpu.make_async_copy(xb.at[cs], o_hbm.at[c], sox.at[cs]).start()
        pltpu.make_async_copy(nb.at[cs], no_hbm.at[c], son.at[cs]).start()

    @pl.when(i == nchunks + 1)
    def _final_drain():
        for k in range(_NSLOT):
            cc = nchunks - _NSLOT + k
            pltpu.make_async_copy(xb.at[k], o_hbm.at[cc], sox.at[k]).wait()
            pltpu.make_async_copy(nb.at[k], no_hbm.at[cc], son.at[k]).wait()


def kernel(clean_future, timesteps, noise):
    batch, ch, h, w = clean_future.shape

    grid_spec = pltpu.PrefetchScalarGridSpec(
        num_scalar_prefetch=3,
        grid=(batch + 2,),
        in_specs=[
            pl.BlockSpec(memory_space=pltpu.MemorySpace.HBM),
            pl.BlockSpec(memory_space=pltpu.MemorySpace.HBM),
        ],
        out_specs=[
            pl.BlockSpec(memory_space=pltpu.MemorySpace.HBM),
            pl.BlockSpec(memory_space=pltpu.MemorySpace.HBM),
        ],
        scratch_shapes=[
            pltpu.VMEM((_NSLOT, ch, h, w), jnp.float32),
            pltpu.VMEM((_NSLOT, ch, h, w), jnp.float32),
            pltpu.SemaphoreType.DMA((_NSLOT,)),
            pltpu.SemaphoreType.DMA((_NSLOT,)),
            pltpu.SemaphoreType.DMA((_NSLOT,)),
            pltpu.SemaphoreType.DMA((_NSLOT,)),
        ],
    )

    out, n_out = pl.pallas_call(
        _body,
        grid_spec=grid_spec,
        out_shape=[
            jax.ShapeDtypeStruct(clean_future.shape, jnp.float32),
            jax.ShapeDtypeStruct(clean_future.shape, jnp.float32),
        ],
    )(timesteps, jnp.asarray(_SQRT_ACP), jnp.asarray(_SQRT_OMACP),
      clean_future, noise)

    return out, n_out


# speedup vs baseline: 23.5701x; 1.0710x over previous
import numpy as np
import jax
import jax.numpy as jnp
from jax.experimental import pallas as pl
from jax.experimental.pallas import tpu as pltpu

_DIFFUSION_STEPS = 1000
_BETA_START = 0.0001
_BETA_END = 0.02


def _make_tables():
    betas = np.linspace(_BETA_START, _BETA_END, _DIFFUSION_STEPS, dtype=np.float32)
    alphas = (np.float32(1.0) - betas).astype(np.float32)
    alphas_cumprod = np.cumprod(alphas, dtype=np.float32)
    sqrt_acp = np.sqrt(alphas_cumprod).astype(np.float32)
    sqrt_omacp = np.sqrt((np.float32(1.0) - alphas_cumprod)).astype(np.float32)
    return sqrt_acp, sqrt_omacp


_SQRT_ACP, _SQRT_OMACP = _make_tables()

_LANES = 128


_SAMPLES_PER_BLOCK = 8


def _noise_body(ts_ref, a_tab_ref, b_tab_ref, x_ref, n_ref, o_ref, n_out_ref):
    i = pl.program_id(0)
    for s in range(_SAMPLES_PER_BLOCK):
        t = ts_ref[i * _SAMPLES_PER_BLOCK + s]
        a = a_tab_ref[t]
        b = b_tab_ref[t]
        nv = n_ref[s]
        o_ref[s] = a * x_ref[s] + b * nv
        n_out_ref[s] = nv


def kernel(clean_future, timesteps, noise):
    batch, ch, h, w = clean_future.shape

    spb = _SAMPLES_PER_BLOCK
    block = (spb, ch, h, w)
    grid_spec = pltpu.PrefetchScalarGridSpec(
        num_scalar_prefetch=3,
        grid=(batch // spb,),
        in_specs=[
            pl.BlockSpec(block, lambda i, *_: (i, 0, 0, 0)),
            pl.BlockSpec(block, lambda i, *_: (i, 0, 0, 0)),
        ],
        out_specs=[
            pl.BlockSpec(block, lambda i, *_: (i, 0, 0, 0)),
            pl.BlockSpec(block, lambda i, *_: (i, 0, 0, 0)),
        ],
    )

    out, n_out = pl.pallas_call(
        _noise_body,
        grid_spec=grid_spec,
        out_shape=[
            jax.ShapeDtypeStruct(clean_future.shape, jnp.float32),
            jax.ShapeDtypeStruct(clean_future.shape, jnp.float32),
        ],
    )(timesteps, jnp.asarray(_SQRT_ACP), jnp.asarray(_SQRT_OMACP), clean_future, noise)

    return out, n_out
